# baseline (speedup 1.0000x reference)
import functools

import jax
import jax.numpy as jnp
from jax import lax
from jax.experimental import pallas as pl
from jax.experimental.pallas import tpu as pltpu
from jax.experimental.pallas import tpu_sc as plsc

EMBED = 64
BATCH = 4096
SEQ = 200
B = BATCH * SEQ
VOCAB = 1000000
TROWS = VOCAB // 2

NSLICE = 4
SEQ_SL = SEQ // NSLICE
ROWS_SL = SEQ_SL * BATCH

NC = 2
NS = 16
NW = NC * NS
CHUNK = 512
N_CHUNKS_SL = ROWS_SL // CHUNK


@functools.lru_cache(maxsize=1)
def _make_sc_gather():
    mesh = plsc.VectorSubcoreMesh(core_axis_name="c", subcore_axis_name="s")

    @functools.partial(
        pl.kernel,
        mesh=mesh,
        out_type=jax.ShapeDtypeStruct((ROWS_SL, 2 * EMBED), jnp.float32),
        scratch_types=[
            pltpu.VMEM((CHUNK,), jnp.int32),
            pltpu.VMEM((CHUNK, 2 * EMBED), jnp.float32),
            pltpu.SemaphoreType.DMA,
        ],
        compiler_params=pltpu.CompilerParams(use_tc_tiling_on_sc=False),
    )
    def _sc_gather(idx_hbm, table_hbm, out_hbm, idx_v, rows_v, sem):
        wid = lax.axis_index("s") * NC + lax.axis_index("c")
        n_w = (N_CHUNKS_SL + NW - 1 - wid) // NW

        def body(i, carry):
            off = (wid + i * NW) * CHUNK
            pltpu.sync_copy(idx_hbm.at[pl.ds(off, CHUNK)], idx_v)
            pltpu.async_copy(table_hbm.at[idx_v], rows_v, sem).wait()
            pltpu.sync_copy(rows_v, out_hbm.at[pl.ds(off, CHUNK)])
            return carry

        lax.fori_loop(0, n_w, body, 0)

    return _sc_gather


S_BLK = 2


def _ln_t_body(y_ref, par_ref, pos_ref, gamma_ref, beta_ref, *refs):
    out_ref = refs[-1]
    g = gamma_ref[...]
    bta = beta_ref[...]
    for si in range(S_BLK):
        xt = y_ref[si * BATCH:(si + 1) * BATCH, :].T
        odd = (par_ref[si, 0] == 1)[None, :]
        x = jnp.where(odd, xt[EMBED:, :], xt[:EMBED, :])
        x = x + pos_ref[si]
        mean = jnp.mean(x, axis=0, keepdims=True)
        xc = x - mean
        var = jnp.mean(xc * xc, axis=0, keepdims=True)
        out_ref[si] = xc * lax.rsqrt(var + 1e-5) * g + bta


def _ln_pallas(k, y, par, pos3, g64, b64, prev=None, interpret=False):
    blk0 = k * (SEQ_SL // S_BLK)
    in_specs = [
        pl.BlockSpec((S_BLK * BATCH, 2 * EMBED), lambda i: (i, 0)),
        pl.BlockSpec((S_BLK, 1, BATCH), lambda i: (i, 0, 0)),
        pl.BlockSpec((S_BLK, EMBED, 1), lambda i: (i, 0, 0)),
        pl.BlockSpec((EMBED, 1), lambda i: (0, 0)),
        pl.BlockSpec((EMBED, 1), lambda i: (0, 0)),
    ]
    args = [y, par, pos3, g64, b64]
    aliases = {}
    if prev is not None:
        in_specs.append(pl.BlockSpec(memory_space=pl.ANY))
        args.append(prev)
        aliases = {5: 0}
    return pl.pallas_call(
        _ln_t_body,
        grid=(SEQ_SL // S_BLK,),
        in_specs=in_specs,
        out_specs=pl.BlockSpec((S_BLK, EMBED, BATCH),
                               lambda i: (blk0 + i, 0, 0)),
        out_shape=jax.ShapeDtypeStruct((SEQ, EMBED, BATCH), jnp.float32),
        input_output_aliases=aliases,
        interpret=interpret,
    )(*args)


def kernel(input_tokens, token_table, position_table, ln_gamma, ln_beta):
    tok_sm = input_tokens.T.astype(jnp.int32).reshape(B)
    idx = tok_sm // 2
    par3 = (tok_sm % 2).reshape(SEQ, 1, BATCH)
    table128 = token_table.reshape(TROWS, 2 * EMBED)
    pos3 = position_table.reshape(SEQ, EMBED, 1)
    g64 = ln_gamma.reshape(EMBED, 1)
    b64 = ln_beta.reshape(EMBED, 1)

    sc_gather = _make_sc_gather()
    gathered = [
        sc_gather(idx[k * ROWS_SL:(k + 1) * ROWS_SL], table128)
        for k in range(NSLICE)
    ]
    out3 = None
    for k in range(NSLICE):
        par_k = par3[k * SEQ_SL:(k + 1) * SEQ_SL]
        pos_k = pos3[k * SEQ_SL:(k + 1) * SEQ_SL]
        out3 = _ln_pallas(k, gathered[k], par_k, pos_k, g64, b64, prev=out3)
    return out3.transpose(2, 0, 1)


# speedup vs baseline: 1.0604x; 1.0604x over previous
import functools

import jax
import jax.numpy as jnp
from jax import lax
from jax.experimental import pallas as pl
from jax.experimental.pallas import tpu as pltpu
from jax.experimental.pallas import tpu_sc as plsc

EMBED = 64
BATCH = 4096
SEQ = 200
B = BATCH * SEQ

NC = 2
NS = 16
NW = NC * NS
B_PER_W = B // NW
CHUNK = 1024
N_CHUNKS = B_PER_W // CHUNK

@functools.lru_cache(maxsize=1)
def _make_sc_gather():
    mesh = plsc.VectorSubcoreMesh(core_axis_name="c", subcore_axis_name="s")

    @functools.partial(
        pl.kernel,
        mesh=mesh,
        out_type=jax.ShapeDtypeStruct((B, EMBED), jnp.float32),
        scratch_types=[
            pltpu.VMEM((CHUNK,), jnp.int32),
            pltpu.VMEM((CHUNK, EMBED), jnp.float32),
            pltpu.SemaphoreType.DMA,
        ],
        compiler_params=pltpu.CompilerParams(use_tc_tiling_on_sc=False),
    )
    def _sc_gather(tok_hbm, table_hbm, out_hbm, idx_v, rows_v, sem):
        wid = lax.axis_index("s") * NC + lax.axis_index("c")
        base = wid * B_PER_W

        def body(i, carry):
            off = base + i * CHUNK
            pltpu.sync_copy(tok_hbm.at[pl.ds(off, CHUNK)], idx_v)
            pltpu.async_copy(table_hbm.at[idx_v], rows_v, sem).wait()
            pltpu.sync_copy(rows_v, out_hbm.at[pl.ds(off, CHUNK)])
            return carry

        lax.fori_loop(0, N_CHUNKS, body, 0)

    return _sc_gather


HALF = BATCH // 2


S_BLK = 8


def _ln_t_body(y_ref, pos_ref, gamma_ref, beta_ref, out_ref):
    g = gamma_ref[...]
    bta = beta_ref[...]
    for si in range(S_BLK):
        x = y_ref[si * HALF:(si + 1) * HALF, :] + pos_ref[si, 0]
        for h in (0, 1):
            t = x[:, h * EMBED:(h + 1) * EMBED].T
            mean = jnp.mean(t, axis=0, keepdims=True)
            tc = t - mean
            var = jnp.mean(tc * tc, axis=0, keepdims=True)
            yh = tc * lax.rsqrt(var + 1e-5) * g + bta
            out_ref[si, :, h * HALF:(h + 1) * HALF] = yh


def _ln_pallas(y, pos128, g64, b64, interpret=False):
    return pl.pallas_call(
        _ln_t_body,
        grid=(SEQ // S_BLK,),
        in_specs=[
            pl.BlockSpec((S_BLK * HALF, 2 * EMBED), lambda i: (i, 0)),
            pl.BlockSpec((S_BLK, 1, 2 * EMBED), lambda i: (i, 0, 0)),
            pl.BlockSpec((EMBED, 1), lambda i: (0, 0)),
            pl.BlockSpec((EMBED, 1), lambda i: (0, 0)),
        ],
        out_specs=pl.BlockSpec((S_BLK, EMBED, BATCH), lambda i: (i, 0, 0)),
        out_shape=jax.ShapeDtypeStruct((SEQ, EMBED, BATCH), jnp.float32),
        interpret=interpret,
    )(y, pos128, g64, b64)


def kernel(input_tokens, token_table, position_table, ln_gamma, ln_beta):
    tok_perm = (
        input_tokens.T.astype(jnp.int32)
        .reshape(SEQ, 2, HALF)
        .transpose(0, 2, 1)
        .reshape(B)
    )
    gathered = _make_sc_gather()(tok_perm, token_table)
    y = gathered.reshape(B // 2, 2 * EMBED)

    pos128 = jnp.concatenate([position_table, position_table], axis=1).reshape(
        SEQ, 1, 2 * EMBED
    )
    g64 = ln_gamma.reshape(EMBED, 1)
    b64 = ln_beta.reshape(EMBED, 1)

    out3 = _ln_pallas(y, pos128, g64, b64)
    return out3.transpose(2, 0, 1)


# speedup vs baseline: 1.0642x; 1.0036x over previous
import functools

import jax
import jax.numpy as jnp
from jax import lax
from jax.experimental import pallas as pl
from jax.experimental.pallas import tpu as pltpu
from jax.experimental.pallas import tpu_sc as plsc

EMBED = 64
BATCH = 4096
SEQ = 200
B = BATCH * SEQ

NC = 2
NS = 16
NW = NC * NS
B_PER_W = B // NW
CHUNK = 512
N_CHUNKS = B_PER_W // CHUNK

@functools.lru_cache(maxsize=1)
def _make_sc_gather():
    mesh = plsc.VectorSubcoreMesh(core_axis_name="c", subcore_axis_name="s")

    @functools.partial(
        pl.kernel,
        mesh=mesh,
        out_type=jax.ShapeDtypeStruct((B, EMBED), jnp.float32),
        scratch_types=[
            pltpu.VMEM((CHUNK,), jnp.int32),
            pltpu.VMEM((CHUNK, EMBED), jnp.float32),
            pltpu.VMEM((CHUNK, EMBED), jnp.float32),
            pltpu.SemaphoreType.DMA,
            pltpu.SemaphoreType.DMA,
            pltpu.SemaphoreType.DMA,
        ],
        compiler_params=pltpu.CompilerParams(use_tc_tiling_on_sc=False),
    )
    def _sc_gather(tok_hbm, table_hbm, out_hbm, idx_v, rows_v0, rows_v1,
                   semg, semw0, semw1):
        wid = lax.axis_index("s") * NC + lax.axis_index("c")
        base = wid * B_PER_W
        rows = (rows_v0, rows_v1)
        semw = (semw0, semw1)

        def body(i, carry):
            for p in (0, 1):
                c = i * 2 + p
                off = base + c * CHUNK

                @pl.when(i > 0)
                def _drain():
                    pltpu.make_async_copy(
                        rows[p],
                        out_hbm.at[pl.ds(off - 2 * CHUNK, CHUNK)],
                        semw[p],
                    ).wait()

                pltpu.sync_copy(tok_hbm.at[pl.ds(off, CHUNK)], idx_v)
                pltpu.async_copy(table_hbm.at[idx_v], rows[p], semg).wait()
                pltpu.async_copy(rows[p], out_hbm.at[pl.ds(off, CHUNK)],
                                 semw[p])
            return carry

        n2 = N_CHUNKS // 2
        lax.fori_loop(0, n2, body, 0)
        for p in (0, 1):
            last = base + (2 * (n2 - 1) + p) * CHUNK
            pltpu.make_async_copy(
                rows[p], out_hbm.at[pl.ds(last, CHUNK)], semw[p]
            ).wait()

    return _sc_gather


HALF = BATCH // 2


S_BLK = 8


def _ln_t_body(y_ref, pos_ref, gamma_ref, beta_ref, out_ref):
    g = gamma_ref[...]
    bta = beta_ref[...]
    for si in range(S_BLK):
        x = y_ref[si * HALF:(si + 1) * HALF, :] + pos_ref[si, 0]
        for h in (0, 1):
            t = x[:, h * EMBED:(h + 1) * EMBED].T
            mean = jnp.mean(t, axis=0, keepdims=True)
            tc = t - mean
            var = jnp.mean(tc * tc, axis=0, keepdims=True)
            yh = tc * lax.rsqrt(var + 1e-5) * g + bta
            out_ref[si, :, h * HALF:(h + 1) * HALF] = yh


def _ln_pallas(y, pos128, g64, b64, interpret=False):
    return pl.pallas_call(
        _ln_t_body,
        grid=(SEQ // S_BLK,),
        in_specs=[
            pl.BlockSpec((S_BLK * HALF, 2 * EMBED), lambda i: (i, 0)),
            pl.BlockSpec((S_BLK, 1, 2 * EMBED), lambda i: (i, 0, 0)),
            pl.BlockSpec((EMBED, 1), lambda i: (0, 0)),
            pl.BlockSpec((EMBED, 1), lambda i: (0, 0)),
        ],
        out_specs=pl.BlockSpec((S_BLK, EMBED, BATCH), lambda i: (i, 0, 0)),
        out_shape=jax.ShapeDtypeStruct((SEQ, EMBED, BATCH), jnp.float32),
        interpret=interpret,
    )(y, pos128, g64, b64)


def kernel(input_tokens, token_table, position_table, ln_gamma, ln_beta):
    tok_perm = (
        input_tokens.T.astype(jnp.int32)
        .reshape(SEQ, 2, HALF)
        .transpose(0, 2, 1)
        .reshape(B)
    )
    gathered = _make_sc_gather()(tok_perm, token_table)
    y = gathered.reshape(B // 2, 2 * EMBED)

    pos128 = jnp.concatenate([position_table, position_table], axis=1).reshape(
        SEQ, 1, 2 * EMBED
    )
    g64 = ln_gamma.reshape(EMBED, 1)
    b64 = ln_beta.reshape(EMBED, 1)

    out3 = _ln_pallas(y, pos128, g64, b64)
    return out3.transpose(2, 0, 1)
